# baseline (speedup 1.0000x reference)
import functools

import jax
import jax.numpy as jnp
from jax import lax
from jax.experimental import pallas as pl
from jax.experimental.pallas import tpu as pltpu
from jax.experimental.pallas import tpu_sc as plsc

N = 10000
D = 128
E = 320000

NC = 2
NS = 16
NW = NC * NS

CHUNK = 128
NCH = 80
NCH0 = 152
NCH1 = 2 * NCH - NCH0
EPT = NCH * CHUNK
EPAD = NW * EPT

ACC_ROWS = 10240
RPT = ACC_ROWS // NS

DEG_ACC = 10240
DEG_PT = DEG_ACC // NS

_mesh = plsc.VectorSubcoreMesh(
    core_axis_name="c", subcore_axis_name="s", num_cores=NC, num_subcores=NS)


@functools.partial(
    pl.kernel,
    out_type=jax.ShapeDtypeStruct((NC, DEG_ACC), jnp.float32),
    mesh=_mesh,
    scratch_types=[
        pltpu.VMEM((NCH, CHUNK), jnp.int32),
        pltpu.VMEM((CHUNK,), jnp.float32),
        pltpu.VMEM((DEG_PT,), jnp.float32),
        pltpu.VMEM_SHARED((DEG_ACC,), jnp.float32),
    ],
)
def _deg_kernel(dst_hbm, out_hbm, dst_v, ones_v, zeros_v, acc):
  c = lax.axis_index("c")
  s = lax.axis_index("s")
  t = c * NS + s

  def fill(i, carry):
    o = pl.multiple_of(i * 16, 16)
    ones_v[pl.ds(o, 16)] = jnp.full((16,), 1.0, jnp.float32)
    return carry
  lax.fori_loop(0, CHUNK // 16, fill, 0)

  def zfill(i, carry):
    o = pl.multiple_of(i * 16, 16)
    zeros_v[pl.ds(o, 16)] = jnp.zeros((16,), jnp.float32)
    return carry
  lax.fori_loop(0, DEG_PT // 16, zfill, 0)

  pltpu.sync_copy(zeros_v, acc.at[pl.ds(s * DEG_PT, DEG_PT)])
  plsc.subcore_barrier()

  pltpu.sync_copy(dst_hbm.at[pl.ds(t * NCH, NCH)], dst_v)

  def body(j, carry):
    pltpu.sync_copy(ones_v, acc.at[dst_v.at[j]], add=True)
    return carry
  lax.fori_loop(0, NCH, body, 0)

  plsc.subcore_barrier()
  pltpu.sync_copy(acc.at[pl.ds(s * DEG_PT, DEG_PT)],
                  out_hbm.at[c].at[pl.ds(s * DEG_PT, DEG_PT)])


@functools.partial(
    pl.kernel,
    out_type=jax.ShapeDtypeStruct((NC, ACC_ROWS, D), jnp.float32),
    mesh=_mesh,
    scratch_types=[
        pltpu.VMEM((CHUNK,), jnp.int32),
        pltpu.VMEM((CHUNK,), jnp.int32),
        pltpu.VMEM((CHUNK,), jnp.int32),
        pltpu.VMEM((CHUNK,), jnp.int32),
        pltpu.VMEM((CHUNK,), jnp.int32),
        pltpu.VMEM((CHUNK,), jnp.int32),
        pltpu.VMEM((CHUNK,), jnp.int32),
        pltpu.VMEM((CHUNK,), jnp.int32),
        pltpu.VMEM((CHUNK, D), jnp.float32),
        pltpu.VMEM((CHUNK, D), jnp.float32),
        pltpu.VMEM_SHARED((ACC_ROWS, D), jnp.float32),
        pltpu.SemaphoreType.DMA,
        pltpu.SemaphoreType.DMA,
        pltpu.SemaphoreType.DMA,
        pltpu.SemaphoreType.DMA,
        pltpu.SemaphoreType.DMA,
        pltpu.SemaphoreType.DMA,
    ],
)
def _agg_kernel(g_hbm, src_hbm, dst_hbm, out_hbm,
                sb0, sb1, sb2, sb3, db0, db1, db2, db3, rows0, rows1, acc,
                si0, si1, si2, si3, sg0, sg1):
  c = lax.axis_index("c")
  s = lax.axis_index("s")
  sb = (sb0, sb1, sb2, sb3)
  db = (db0, db1, db2, db3)
  si = (si0, si1, si2, si3)
  rows = (rows0, rows1)
  sg = (sg0, sg1)

  def zrow(i, carry):
    for j in range(D // 16):
      rows0[i, pl.ds(j * 16, 16)] = jnp.zeros((16,), jnp.float32)
    return carry
  lax.fori_loop(0, CHUNK, zrow, 0)

  base = s * RPT
  for k in range(RPT // CHUNK):
    pltpu.sync_copy(rows0, acc.at[pl.ds(base + k * CHUNK, CHUNK)])
  plsc.subcore_barrier()

  ncz = jnp.where(c == 0, NCH0, NCH1)
  brow = jnp.where(c == 0, s * NCH0, NS * NCH0 + s * NCH1)

  def idx_start(j, b):
    pltpu.async_copy(src_hbm.at[brow + j], sb[b], si[b])
    pltpu.async_copy(dst_hbm.at[brow + j], db[b], si[b])

  def idx_wait(j, b):
    pltpu.make_async_copy(src_hbm.at[brow + j], sb[b], si[b]).wait()
    pltpu.make_async_copy(dst_hbm.at[brow + j], db[b], si[b]).wait()

  def gat_start(b4, b2):
    pltpu.async_copy(g_hbm.at[sb[b4]], rows[b2], sg[b2])

  def gat_wait(b4, b2):
    pltpu.make_async_copy(g_hbm.at[sb[b4]], rows[b2], sg[b2]).wait()

  def scat(b4, b2):
    pltpu.sync_copy(rows[b2], acc.at[db[b4]], add=True)

  idx_start(0, 0)
  idx_start(1, 1)
  idx_wait(0, 0)
  gat_start(0, 0)

  def body(i, carry):
    jj = i * 4
    for b in range(4):
      j = jj + b
      idx_start(j + 2, (b + 2) % 4)
      idx_wait(j + 1, (b + 1) % 4)
      gat_wait(b, b % 2)
      gat_start((b + 1) % 4, (b + 1) % 2)
      scat(b, b % 2)
    return carry
  lax.fori_loop(0, ncz // 4 - 1, body, 0)

  jt = ncz - 4
  idx_start(jt + 2, 2)
  idx_wait(jt + 1, 1)
  gat_wait(0, 0)
  gat_start(1, 1)
  scat(0, 0)

  idx_start(jt + 3, 3)
  idx_wait(jt + 2, 2)
  gat_wait(1, 1)
  gat_start(2, 0)
  scat(1, 1)

  idx_wait(jt + 3, 3)
  gat_wait(2, 0)
  gat_start(3, 1)
  scat(2, 0)

  gat_wait(3, 1)
  scat(3, 1)

  plsc.subcore_barrier()
  pltpu.sync_copy(acc.at[pl.ds(base, RPT)],
                  out_hbm.at[c].at[pl.ds(base, RPT)])


_RB = 1000


def _prep_body(x_ref, deg_ref, w_ref, dinv_ref, g_ref):
  deg = deg_ref[:, 0:1] + deg_ref[:, 1:2] + 1.0
  dinv = lax.rsqrt(deg)
  dinv_ref[...] = dinv
  h = jnp.dot(x_ref[...], w_ref[...], preferred_element_type=jnp.float32)
  g_ref[...] = h * dinv


def _prep(x, deg_t, w0):
  return pl.pallas_call(
      _prep_body,
      grid=(N // _RB,),
      in_specs=[
          pl.BlockSpec((_RB, D), lambda i: (i, 0)),
          pl.BlockSpec((_RB, NC), lambda i: (i, 0)),
          pl.BlockSpec((D, D), lambda i: (0, 0)),
      ],
      out_specs=[
          pl.BlockSpec((_RB, 1), lambda i: (i, 0)),
          pl.BlockSpec((_RB, D), lambda i: (i, 0)),
      ],
      out_shape=[
          jax.ShapeDtypeStruct((N, 1), jnp.float32),
          jax.ShapeDtypeStruct((N, D), jnp.float32),
      ],
  )(x, deg_t, w0)


_CB = ACC_ROWS // 8


def _combine_body(p_ref, g_ref, dinv_ref, b_ref, w_ref, out_ref, gn_ref):
  ssum = p_ref[0] + p_ref[1] + g_ref[...]
  dinv = dinv_ref[...]
  out = dinv * ssum + b_ref[...]
  out_ref[...] = out
  gn_ref[...] = jnp.dot(out, w_ref[...],
                        preferred_element_type=jnp.float32) * dinv


def _combine(p, g, dinv, b, wn):
  return pl.pallas_call(
      _combine_body,
      grid=(8,),
      in_specs=[
          pl.BlockSpec((NC, _CB, D), lambda i: (0, i, 0)),
          pl.BlockSpec((_CB, D), lambda i: (i, 0)),
          pl.BlockSpec((_CB, 1), lambda i: (i, 0)),
          pl.BlockSpec((1, D), lambda i: (0, 0)),
          pl.BlockSpec((D, D), lambda i: (0, 0)),
      ],
      out_specs=[
          pl.BlockSpec((_CB, D), lambda i: (i, 0)),
          pl.BlockSpec((_CB, D), lambda i: (i, 0)),
      ],
      out_shape=[
          jax.ShapeDtypeStruct((N, D), jnp.float32),
          jax.ShapeDtypeStruct((N, D), jnp.float32),
      ],
  )(p, g, dinv, b, wn)


def kernel(x, edge_index, W0, b0, W1, b1, W2, b2):
  src = edge_index[0].astype(jnp.int32)
  dst = edge_index[1].astype(jnp.int32)

  pad = EPAD - E
  src_p = jnp.concatenate([src, jnp.zeros((pad,), jnp.int32)])
  dst_p = jnp.concatenate([dst, jnp.full((pad,), N, jnp.int32)])
  src2d = src_p.reshape(NW * NCH, CHUNK)
  dst2d = dst_p.reshape(NW * NCH, CHUNK)

  deg = _deg_kernel(dst2d)
  dinv, g = _prep(x, deg.T, W0)

  outs = [x]
  for (b, wn) in ((b0, W1), (b1, W2), (b2, W2)):
    p = _agg_kernel(g, src2d, dst2d)
    out, g = _combine(p, g, dinv, b.reshape(1, D), wn)
    outs.append(out)

  return jnp.concatenate(outs, axis=-1)


# speedup vs baseline: 1.0060x; 1.0060x over previous
import functools

import jax
import jax.numpy as jnp
from jax import lax
from jax.experimental import pallas as pl
from jax.experimental.pallas import tpu as pltpu
from jax.experimental.pallas import tpu_sc as plsc

N = 10000
D = 128
E = 320000

NC = 2
NS = 16
NW = NC * NS

CHUNK = 128
NCH = 80
NCH0 = 144
NCH1 = 2 * NCH - NCH0
EPT = NCH * CHUNK
EPAD = NW * EPT

ACC_ROWS = 10240
RPT = ACC_ROWS // NS

DEG_ACC = 10240
DEG_PT = DEG_ACC // NS

_mesh = plsc.VectorSubcoreMesh(
    core_axis_name="c", subcore_axis_name="s", num_cores=NC, num_subcores=NS)


@functools.partial(
    pl.kernel,
    out_type=jax.ShapeDtypeStruct((NC, DEG_ACC), jnp.float32),
    mesh=_mesh,
    scratch_types=[
        pltpu.VMEM((NCH, CHUNK), jnp.int32),
        pltpu.VMEM((CHUNK,), jnp.float32),
        pltpu.VMEM((DEG_PT,), jnp.float32),
        pltpu.VMEM_SHARED((DEG_ACC,), jnp.float32),
    ],
)
def _deg_kernel(dst_hbm, out_hbm, dst_v, ones_v, zeros_v, acc):
  c = lax.axis_index("c")
  s = lax.axis_index("s")
  t = c * NS + s

  def fill(i, carry):
    o = pl.multiple_of(i * 16, 16)
    ones_v[pl.ds(o, 16)] = jnp.full((16,), 1.0, jnp.float32)
    return carry
  lax.fori_loop(0, CHUNK // 16, fill, 0)

  def zfill(i, carry):
    o = pl.multiple_of(i * 16, 16)
    zeros_v[pl.ds(o, 16)] = jnp.zeros((16,), jnp.float32)
    return carry
  lax.fori_loop(0, DEG_PT // 16, zfill, 0)

  pltpu.sync_copy(zeros_v, acc.at[pl.ds(s * DEG_PT, DEG_PT)])
  plsc.subcore_barrier()

  pltpu.sync_copy(dst_hbm.at[pl.ds(t * NCH, NCH)], dst_v)

  def body(j, carry):
    pltpu.sync_copy(ones_v, acc.at[dst_v.at[j]], add=True)
    return carry
  lax.fori_loop(0, NCH, body, 0)

  plsc.subcore_barrier()
  pltpu.sync_copy(acc.at[pl.ds(s * DEG_PT, DEG_PT)],
                  out_hbm.at[c].at[pl.ds(s * DEG_PT, DEG_PT)])


@functools.partial(
    pl.kernel,
    out_type=jax.ShapeDtypeStruct((NC, ACC_ROWS, D), jnp.float32),
    mesh=_mesh,
    scratch_types=[
        pltpu.VMEM((CHUNK,), jnp.int32),
        pltpu.VMEM((CHUNK,), jnp.int32),
        pltpu.VMEM((CHUNK,), jnp.int32),
        pltpu.VMEM((CHUNK,), jnp.int32),
        pltpu.VMEM((CHUNK,), jnp.int32),
        pltpu.VMEM((CHUNK,), jnp.int32),
        pltpu.VMEM((CHUNK,), jnp.int32),
        pltpu.VMEM((CHUNK,), jnp.int32),
        pltpu.VMEM((CHUNK, D), jnp.float32),
        pltpu.VMEM((CHUNK, D), jnp.float32),
        pltpu.VMEM_SHARED((ACC_ROWS, D), jnp.float32),
        pltpu.SemaphoreType.DMA,
        pltpu.SemaphoreType.DMA,
        pltpu.SemaphoreType.DMA,
        pltpu.SemaphoreType.DMA,
        pltpu.SemaphoreType.DMA,
        pltpu.SemaphoreType.DMA,
    ],
)
def _agg_kernel(g_hbm, src_hbm, dst_hbm, out_hbm,
                sb0, sb1, sb2, sb3, db0, db1, db2, db3, rows0, rows1, acc,
                si0, si1, si2, si3, sg0, sg1):
  c = lax.axis_index("c")
  s = lax.axis_index("s")
  sb = (sb0, sb1, sb2, sb3)
  db = (db0, db1, db2, db3)
  si = (si0, si1, si2, si3)
  rows = (rows0, rows1)
  sg = (sg0, sg1)

  def zrow(i, carry):
    for j in range(D // 16):
      rows0[i, pl.ds(j * 16, 16)] = jnp.zeros((16,), jnp.float32)
    return carry
  lax.fori_loop(0, CHUNK, zrow, 0)

  base = s * RPT
  for k in range(RPT // CHUNK):
    pltpu.sync_copy(rows0, acc.at[pl.ds(base + k * CHUNK, CHUNK)])
  plsc.subcore_barrier()

  ncz = jnp.where(c == 0, NCH0, NCH1)
  brow = jnp.where(c == 0, s * NCH0, NS * NCH0 + s * NCH1)

  def idx_start(j, b):
    pltpu.async_copy(src_hbm.at[brow + j], sb[b], si[b])
    pltpu.async_copy(dst_hbm.at[brow + j], db[b], si[b])

  def idx_wait(j, b):
    pltpu.make_async_copy(src_hbm.at[brow + j], sb[b], si[b]).wait()
    pltpu.make_async_copy(dst_hbm.at[brow + j], db[b], si[b]).wait()

  def gat_start(b4, b2):
    pltpu.async_copy(g_hbm.at[sb[b4]], rows[b2], sg[b2])

  def gat_wait(b4, b2):
    pltpu.make_async_copy(g_hbm.at[sb[b4]], rows[b2], sg[b2]).wait()

  def scat(b4, b2):
    pltpu.sync_copy(rows[b2], acc.at[db[b4]], add=True)

  idx_start(0, 0)
  idx_start(1, 1)
  idx_wait(0, 0)
  gat_start(0, 0)

  def body(i, carry):
    jj = i * 4
    for b in range(4):
      j = jj + b
      idx_start(j + 2, (b + 2) % 4)
      idx_wait(j + 1, (b + 1) % 4)
      gat_wait(b, b % 2)
      gat_start((b + 1) % 4, (b + 1) % 2)
      scat(b, b % 2)
    return carry
  lax.fori_loop(0, ncz // 4 - 1, body, 0)

  jt = ncz - 4
  idx_start(jt + 2, 2)
  idx_wait(jt + 1, 1)
  gat_wait(0, 0)
  gat_start(1, 1)
  scat(0, 0)

  idx_start(jt + 3, 3)
  idx_wait(jt + 2, 2)
  gat_wait(1, 1)
  gat_start(2, 0)
  scat(1, 1)

  idx_wait(jt + 3, 3)
  gat_wait(2, 0)
  gat_start(3, 1)
  scat(2, 0)

  gat_wait(3, 1)
  scat(3, 1)

  plsc.subcore_barrier()
  pltpu.sync_copy(acc.at[pl.ds(base, RPT)],
                  out_hbm.at[c].at[pl.ds(base, RPT)])


_RB = 1000


def _prep_body(x_ref, deg_ref, w_ref, dinv_ref, g_ref):
  deg = deg_ref[:, 0:1] + deg_ref[:, 1:2] + 1.0
  dinv = lax.rsqrt(deg)
  dinv_ref[...] = dinv
  h = jnp.dot(x_ref[...], w_ref[...], preferred_element_type=jnp.float32)
  g_ref[...] = h * dinv


def _prep(x, deg_t, w0):
  return pl.pallas_call(
      _prep_body,
      grid=(N // _RB,),
      in_specs=[
          pl.BlockSpec((_RB, D), lambda i: (i, 0)),
          pl.BlockSpec((_RB, NC), lambda i: (i, 0)),
          pl.BlockSpec((D, D), lambda i: (0, 0)),
      ],
      out_specs=[
          pl.BlockSpec((_RB, 1), lambda i: (i, 0)),
          pl.BlockSpec((_RB, D), lambda i: (i, 0)),
      ],
      out_shape=[
          jax.ShapeDtypeStruct((N, 1), jnp.float32),
          jax.ShapeDtypeStruct((N, D), jnp.float32),
      ],
  )(x, deg_t, w0)


_CB = ACC_ROWS // 8


def _combine_body(p_ref, g_ref, dinv_ref, b_ref, w_ref, out_ref, gn_ref):
  ssum = p_ref[0] + p_ref[1] + g_ref[...]
  dinv = dinv_ref[...]
  out = dinv * ssum + b_ref[...]
  out_ref[...] = out
  gn_ref[...] = jnp.dot(out, w_ref[...],
                        preferred_element_type=jnp.float32) * dinv


def _combine(p, g, dinv, b, wn):
  return pl.pallas_call(
      _combine_body,
      grid=(8,),
      in_specs=[
          pl.BlockSpec((NC, _CB, D), lambda i: (0, i, 0)),
          pl.BlockSpec((_CB, D), lambda i: (i, 0)),
          pl.BlockSpec((_CB, 1), lambda i: (i, 0)),
          pl.BlockSpec((1, D), lambda i: (0, 0)),
          pl.BlockSpec((D, D), lambda i: (0, 0)),
      ],
      out_specs=[
          pl.BlockSpec((_CB, D), lambda i: (i, 0)),
          pl.BlockSpec((_CB, D), lambda i: (i, 0)),
      ],
      out_shape=[
          jax.ShapeDtypeStruct((N, D), jnp.float32),
          jax.ShapeDtypeStruct((N, D), jnp.float32),
      ],
  )(p, g, dinv, b, wn)


def kernel(x, edge_index, W0, b0, W1, b1, W2, b2):
  src = edge_index[0].astype(jnp.int32)
  dst = edge_index[1].astype(jnp.int32)

  pad = EPAD - E
  src_p = jnp.concatenate([src, jnp.zeros((pad,), jnp.int32)])
  dst_p = jnp.concatenate([dst, jnp.full((pad,), N, jnp.int32)])
  src2d = src_p.reshape(NW * NCH, CHUNK)
  dst2d = dst_p.reshape(NW * NCH, CHUNK)

  deg = _deg_kernel(dst2d)
  dinv, g = _prep(x, deg.T, W0)

  outs = [x]
  for (b, wn) in ((b0, W1), (b1, W2), (b2, W2)):
    p = _agg_kernel(g, src2d, dst2d)
    out, g = _combine(p, g, dinv, b.reshape(1, D), wn)
    outs.append(out)

  return jnp.concatenate(outs, axis=-1)


# speedup vs baseline: 1.0168x; 1.0108x over previous
import functools

import jax
import jax.numpy as jnp
from jax import lax
from jax.experimental import pallas as pl
from jax.experimental.pallas import tpu as pltpu
from jax.experimental.pallas import tpu_sc as plsc

N = 10000
D = 128
E = 320000

NC = 2
NS = 16
NW = NC * NS

CHUNK = 128
NCH = 80
NCH0 = 148
NCH1 = 2 * NCH - NCH0
EPT = NCH * CHUNK
EPAD = NW * EPT

ACC_ROWS = 10240
RPT = ACC_ROWS // NS

DEG_ACC = 10240
DEG_PT = DEG_ACC // NS

_mesh = plsc.VectorSubcoreMesh(
    core_axis_name="c", subcore_axis_name="s", num_cores=NC, num_subcores=NS)


@functools.partial(
    pl.kernel,
    out_type=jax.ShapeDtypeStruct((NC, DEG_ACC), jnp.float32),
    mesh=_mesh,
    scratch_types=[
        pltpu.VMEM((NCH, CHUNK), jnp.int32),
        pltpu.VMEM((CHUNK,), jnp.float32),
        pltpu.VMEM((DEG_PT,), jnp.float32),
        pltpu.VMEM_SHARED((DEG_ACC,), jnp.float32),
    ],
)
def _deg_kernel(dst_hbm, out_hbm, dst_v, ones_v, zeros_v, acc):
  c = lax.axis_index("c")
  s = lax.axis_index("s")
  t = c * NS + s

  def fill(i, carry):
    o = pl.multiple_of(i * 16, 16)
    ones_v[pl.ds(o, 16)] = jnp.full((16,), 1.0, jnp.float32)
    return carry
  lax.fori_loop(0, CHUNK // 16, fill, 0)

  def zfill(i, carry):
    o = pl.multiple_of(i * 16, 16)
    zeros_v[pl.ds(o, 16)] = jnp.zeros((16,), jnp.float32)
    return carry
  lax.fori_loop(0, DEG_PT // 16, zfill, 0)

  pltpu.sync_copy(zeros_v, acc.at[pl.ds(s * DEG_PT, DEG_PT)])
  plsc.subcore_barrier()

  pltpu.sync_copy(dst_hbm.at[pl.ds(t * NCH, NCH)], dst_v)

  def body(j, carry):
    pltpu.sync_copy(ones_v, acc.at[dst_v.at[j]], add=True)
    return carry
  lax.fori_loop(0, NCH, body, 0)

  plsc.subcore_barrier()
  pltpu.sync_copy(acc.at[pl.ds(s * DEG_PT, DEG_PT)],
                  out_hbm.at[c].at[pl.ds(s * DEG_PT, DEG_PT)])


@functools.partial(
    pl.kernel,
    out_type=jax.ShapeDtypeStruct((NC, ACC_ROWS, D), jnp.float32),
    mesh=_mesh,
    scratch_types=[
        pltpu.VMEM((CHUNK,), jnp.int32),
        pltpu.VMEM((CHUNK,), jnp.int32),
        pltpu.VMEM((CHUNK,), jnp.int32),
        pltpu.VMEM((CHUNK,), jnp.int32),
        pltpu.VMEM((CHUNK,), jnp.int32),
        pltpu.VMEM((CHUNK,), jnp.int32),
        pltpu.VMEM((CHUNK,), jnp.int32),
        pltpu.VMEM((CHUNK,), jnp.int32),
        pltpu.VMEM((CHUNK, D), jnp.float32),
        pltpu.VMEM((CHUNK, D), jnp.float32),
        pltpu.VMEM_SHARED((ACC_ROWS, D), jnp.float32),
        pltpu.SemaphoreType.DMA,
        pltpu.SemaphoreType.DMA,
        pltpu.SemaphoreType.DMA,
        pltpu.SemaphoreType.DMA,
        pltpu.SemaphoreType.DMA,
        pltpu.SemaphoreType.DMA,
    ],
)
def _agg_kernel(g_hbm, src_hbm, dst_hbm, out_hbm,
                sb0, sb1, sb2, sb3, db0, db1, db2, db3, rows0, rows1, acc,
                si0, si1, si2, si3, sg0, sg1):
  c = lax.axis_index("c")
  s = lax.axis_index("s")
  sb = (sb0, sb1, sb2, sb3)
  db = (db0, db1, db2, db3)
  si = (si0, si1, si2, si3)
  rows = (rows0, rows1)
  sg = (sg0, sg1)

  def zrow(i, carry):
    for j in range(D // 16):
      rows0[i, pl.ds(j * 16, 16)] = jnp.zeros((16,), jnp.float32)
    return carry
  lax.fori_loop(0, CHUNK, zrow, 0)

  base = s * RPT
  for k in range(RPT // CHUNK):
    pltpu.sync_copy(rows0, acc.at[pl.ds(base + k * CHUNK, CHUNK)])
  plsc.subcore_barrier()

  ncz = jnp.where(c == 0, NCH0, NCH1)
  brow = jnp.where(c == 0, s * NCH0, NS * NCH0 + s * NCH1)

  def idx_start(j, b):
    pltpu.async_copy(src_hbm.at[brow + j], sb[b], si[b])
    pltpu.async_copy(dst_hbm.at[brow + j], db[b], si[b])

  def idx_wait(j, b):
    pltpu.make_async_copy(src_hbm.at[brow + j], sb[b], si[b]).wait()
    pltpu.make_async_copy(dst_hbm.at[brow + j], db[b], si[b]).wait()

  def gat_start(b4, b2):
    pltpu.async_copy(g_hbm.at[sb[b4]], rows[b2], sg[b2])

  def gat_wait(b4, b2):
    pltpu.make_async_copy(g_hbm.at[sb[b4]], rows[b2], sg[b2]).wait()

  def scat(b4, b2):
    pltpu.sync_copy(rows[b2], acc.at[db[b4]], add=True)

  idx_start(0, 0)
  idx_start(1, 1)
  idx_wait(0, 0)
  gat_start(0, 0)

  def body(i, carry):
    jj = i * 4
    for b in range(4):
      j = jj + b
      idx_start(j + 2, (b + 2) % 4)
      idx_wait(j + 1, (b + 1) % 4)
      gat_wait(b, b % 2)
      gat_start((b + 1) % 4, (b + 1) % 2)
      scat(b, b % 2)
    return carry
  lax.fori_loop(0, ncz // 4 - 1, body, 0)

  jt = ncz - 4
  idx_start(jt + 2, 2)
  idx_wait(jt + 1, 1)
  gat_wait(0, 0)
  gat_start(1, 1)
  scat(0, 0)

  idx_start(jt + 3, 3)
  idx_wait(jt + 2, 2)
  gat_wait(1, 1)
  gat_start(2, 0)
  scat(1, 1)

  idx_wait(jt + 3, 3)
  gat_wait(2, 0)
  gat_start(3, 1)
  scat(2, 0)

  gat_wait(3, 1)
  scat(3, 1)

  plsc.subcore_barrier()
  pltpu.sync_copy(acc.at[pl.ds(base, RPT)],
                  out_hbm.at[c].at[pl.ds(base, RPT)])


_RB = 1000


def _prep_body(x_ref, deg_ref, w_ref, dinv_ref, g_ref):
  deg = deg_ref[:, 0:1] + deg_ref[:, 1:2] + 1.0
  dinv = lax.rsqrt(deg)
  dinv_ref[...] = dinv
  h = jnp.dot(x_ref[...], w_ref[...], preferred_element_type=jnp.float32)
  g_ref[...] = h * dinv


def _prep(x, deg_t, w0):
  return pl.pallas_call(
      _prep_body,
      grid=(N // _RB,),
      in_specs=[
          pl.BlockSpec((_RB, D), lambda i: (i, 0)),
          pl.BlockSpec((_RB, NC), lambda i: (i, 0)),
          pl.BlockSpec((D, D), lambda i: (0, 0)),
      ],
      out_specs=[
          pl.BlockSpec((_RB, 1), lambda i: (i, 0)),
          pl.BlockSpec((_RB, D), lambda i: (i, 0)),
      ],
      out_shape=[
          jax.ShapeDtypeStruct((N, 1), jnp.float32),
          jax.ShapeDtypeStruct((N, D), jnp.float32),
      ],
  )(x, deg_t, w0)


_CB = ACC_ROWS // 8


def _combine_body(p_ref, g_ref, dinv_ref, b_ref, w_ref, out_ref, gn_ref):
  ssum = p_ref[0] + p_ref[1] + g_ref[...]
  dinv = dinv_ref[...]
  out = dinv * ssum + b_ref[...]
  out_ref[...] = out
  gn_ref[...] = jnp.dot(out, w_ref[...],
                        preferred_element_type=jnp.float32) * dinv


def _combine(p, g, dinv, b, wn):
  return pl.pallas_call(
      _combine_body,
      grid=(8,),
      in_specs=[
          pl.BlockSpec((NC, _CB, D), lambda i: (0, i, 0)),
          pl.BlockSpec((_CB, D), lambda i: (i, 0)),
          pl.BlockSpec((_CB, 1), lambda i: (i, 0)),
          pl.BlockSpec((1, D), lambda i: (0, 0)),
          pl.BlockSpec((D, D), lambda i: (0, 0)),
      ],
      out_specs=[
          pl.BlockSpec((_CB, D), lambda i: (i, 0)),
          pl.BlockSpec((_CB, D), lambda i: (i, 0)),
      ],
      out_shape=[
          jax.ShapeDtypeStruct((N, D), jnp.float32),
          jax.ShapeDtypeStruct((N, D), jnp.float32),
      ],
  )(p, g, dinv, b, wn)


def kernel(x, edge_index, W0, b0, W1, b1, W2, b2):
  src = edge_index[0].astype(jnp.int32)
  dst = edge_index[1].astype(jnp.int32)

  pad = EPAD - E
  src_p = jnp.concatenate([src, jnp.zeros((pad,), jnp.int32)])
  dst_p = jnp.concatenate([dst, jnp.full((pad,), N, jnp.int32)])
  src2d = src_p.reshape(NW * NCH, CHUNK)
  dst2d = dst_p.reshape(NW * NCH, CHUNK)

  deg = _deg_kernel(dst2d)
  dinv, g = _prep(x, deg.T, W0)

  outs = [x]
  for (b, wn) in ((b0, W1), (b1, W2), (b2, W2)):
    p = _agg_kernel(g, src2d, dst2d)
    out, g = _combine(p, g, dinv, b.reshape(1, D), wn)
    outs.append(out)

  return jnp.concatenate(outs, axis=-1)
